# baseline (speedup 1.0000x reference)
import functools

import jax
import jax.numpy as jnp
from jax.experimental import pallas as pl
from jax.experimental.pallas import tpu as pltpu


def _fused_kernel(x_hbm, w_ref, adj_hbm, b_ref, o_ref,
                  x_ref, adj_buf, sup_ref, x_sem, adj_sem, *,
                  wgt_alpha, act_alpha, w_levels, a_levels, n_elem,
                  tile, ntiles):
    i = pl.program_id(0)
    rows_per_core = ntiles * tile
    base = i * rows_per_core

    adj_cps = []
    for t in range(ntiles):
        cp = pltpu.make_async_copy(
            adj_hbm.at[pl.ds(base + t * tile, tile), :],
            adj_buf.at[t], adj_sem.at[t])
        cp.start()
        adj_cps.append(cp)
    x_cp = pltpu.make_async_copy(x_hbm, x_ref, x_sem)
    x_cp.start()

    w = w_ref[...]
    mean = jnp.sum(w) / n_elem
    var = jnp.sum((w - mean) ** 2) / (n_elem - 1.0)
    w_n = (w - mean) / jnp.sqrt(var)
    wc = jnp.clip(w_n / wgt_alpha, -1.0, 1.0)
    w_q = (jnp.round(jnp.abs(wc) * w_levels) / w_levels) \
        * jnp.sign(wc) * wgt_alpha

    x_cp.wait()
    xc = jnp.minimum(x_ref[...] / act_alpha, 1.0)
    x_q = (jnp.round(xc * a_levels) / a_levels) * act_alpha
    sup_ref[...] = jnp.dot(
        x_q.astype(jnp.bfloat16), w_q.astype(jnp.bfloat16),
        preferred_element_type=jnp.float32).astype(jnp.bfloat16)

    for t in range(ntiles):
        adj_cps[t].wait()
        o_ref[pl.ds(t * tile, tile), :] = jnp.dot(
            adj_buf[t].astype(jnp.bfloat16), sup_ref[...],
            preferred_element_type=jnp.float32) + b_ref[...]


def kernel(x, adj, weight, bias):
    f32 = jnp.float32
    x = x.astype(f32)
    adj = adj.astype(f32)
    weight = weight.astype(f32)
    n, fin = x.shape
    fout = weight.shape[1]
    b2 = bias.astype(f32).reshape(1, fout)

    cores = 2
    tile = min(1024, n // cores)
    ntiles = (n // cores) // tile
    assert cores * ntiles * tile == n

    out = pl.pallas_call(
        functools.partial(
            _fused_kernel, wgt_alpha=3.0, act_alpha=1.0,
            w_levels=7.0, a_levels=15.0, n_elem=float(fin * fout),
            tile=tile, ntiles=ntiles),
        out_shape=jax.ShapeDtypeStruct((n, fout), f32),
        grid=(cores,),
        in_specs=[
            pl.BlockSpec(memory_space=pl.ANY),
            pl.BlockSpec((fin, fout), lambda i: (0, 0)),
            pl.BlockSpec(memory_space=pl.ANY),
            pl.BlockSpec((1, fout), lambda i: (0, 0)),
        ],
        out_specs=pl.BlockSpec((ntiles * tile, fout), lambda i: (i, 0)),
        scratch_shapes=[
            pltpu.VMEM((n, fin), f32),
            pltpu.VMEM((2, tile, n), f32),
            pltpu.VMEM((n, fout), jnp.bfloat16),
            pltpu.SemaphoreType.DMA,
            pltpu.SemaphoreType.DMA((2,)),
        ],
        compiler_params=pltpu.CompilerParams(
            dimension_semantics=("parallel",),
            vmem_limit_bytes=56 * 1024 * 1024),
        cost_estimate=pl.CostEstimate(
            flops=2 * n * n * fout + 2 * n * fin * fout,
            transcendentals=0,
            bytes_accessed=4 * (n * n + n * fin + fin * fout
                                + n * fout + fout)),
    )(x, weight, adj, b2)
    return out


# speedup vs baseline: 1.2919x; 1.2919x over previous
import functools

import jax
import jax.numpy as jnp
from jax.experimental import pallas as pl
from jax.experimental.pallas import tpu as pltpu


def _fused_kernel(x_ref, w_ref, adj_ref, b_ref, o_ref, sup_ref, *,
                  wgt_alpha, act_alpha, w_levels, a_levels, n_elem):
    @pl.when(pl.program_id(1) == 0)
    def _():
        w = w_ref[...]
        mean = jnp.sum(w) / n_elem
        var = jnp.sum((w - mean) ** 2) / (n_elem - 1.0)
        w_n = (w - mean) / jnp.sqrt(var)
        wc = jnp.clip(w_n / wgt_alpha, -1.0, 1.0)
        w_q = (jnp.round(jnp.abs(wc) * w_levels) / w_levels) \
            * jnp.sign(wc) * wgt_alpha
        xc = jnp.minimum(x_ref[...] / act_alpha, 1.0)
        x_q = (jnp.round(xc * a_levels) / a_levels) * act_alpha
        sup_ref[...] = jnp.dot(
            x_q.astype(jnp.bfloat16), w_q.astype(jnp.bfloat16),
            preferred_element_type=jnp.float32).astype(jnp.bfloat16)

    o_ref[...] = jnp.dot(
        adj_ref[...].astype(jnp.bfloat16), sup_ref[...],
        preferred_element_type=jnp.float32) + b_ref[...]


def kernel(x, adj, weight, bias):
    f32 = jnp.float32
    x = x.astype(f32)
    adj = adj.astype(f32)
    weight = weight.astype(f32)
    n, fin = x.shape
    fout = weight.shape[1]
    b2 = bias.astype(f32).reshape(1, fout)

    cores = 2
    tile = min(1024, n // cores)
    ntiles = (n // cores) // tile
    assert cores * ntiles * tile == n

    out = pl.pallas_call(
        functools.partial(
            _fused_kernel, wgt_alpha=3.0, act_alpha=1.0,
            w_levels=7.0, a_levels=15.0, n_elem=float(fin * fout)),
        out_shape=jax.ShapeDtypeStruct((n, fout), f32),
        grid=(cores, ntiles),
        in_specs=[
            pl.BlockSpec((n, fin), lambda i, j: (0, 0)),
            pl.BlockSpec((fin, fout), lambda i, j: (0, 0)),
            pl.BlockSpec((tile, n), lambda i, j: (i * ntiles + j, 0)),
            pl.BlockSpec((1, fout), lambda i, j: (0, 0)),
        ],
        out_specs=pl.BlockSpec((tile, fout), lambda i, j: (i * ntiles + j, 0)),
        scratch_shapes=[pltpu.VMEM((n, fout), jnp.bfloat16)],
        compiler_params=pltpu.CompilerParams(
            dimension_semantics=("parallel", "arbitrary"),
            vmem_limit_bytes=48 * 1024 * 1024),
        cost_estimate=pl.CostEstimate(
            flops=2 * n * n * fout + 2 * n * fin * fout,
            transcendentals=0,
            bytes_accessed=4 * (n * n + n * fin + fin * fout
                                + n * fout + fout)),
    )(x, weight, adj, b2)
    return out
